# baseline (speedup 1.0000x reference)
import functools

import jax
import jax.numpy as jnp
from jax import lax
from jax.experimental import pallas as pl
from jax.experimental.pallas import tpu as pltpu
from jax.experimental.pallas import tpu_sc as plsc

_NC = 2
_NS = 16
_L = 16
_NW = _NC * _NS

_ROW_CHUNK = 8
_NBUF = 4
_UNROLL = 32


def _make_sc_call(nrows: int, ncols: int, hi: float):
    rows_per_w = nrows // _NW
    n_chunks = rows_per_w // _ROW_CHUNK
    chunk = _ROW_CHUNK * ncols
    mesh = plsc.VectorSubcoreMesh(
        core_axis_name="c", subcore_axis_name="s",
        num_cores=_NC, num_subcores=_NS)

    @functools.partial(
        pl.kernel,
        out_type=jax.ShapeDtypeStruct((nrows, ncols), jnp.float32),
        mesh=mesh,
        compiler_params=pltpu.CompilerParams(
            needs_layout_passes=False, use_tc_tiling_on_sc=True),
        scratch_types=(
            [pltpu.VMEM((32,), jnp.int32)] * 2
            + [pltpu.VMEM((_ROW_CHUNK, ncols), jnp.float32)] * (3 * _NBUF)
            + [pltpu.SemaphoreType.DMA] * (3 * _NBUF)
        ),
    )
    def sc_kernel(x_hbm, e_hbm, pm_hbm, ps_hbm, o_hbm, pm_v, ps_v, *rest):
        bufs = rest[:3 * _NBUF]
        sems = rest[3 * _NBUF:]
        xb, eb, ob = (bufs[:_NBUF], bufs[_NBUF:2 * _NBUF],
                      bufs[2 * _NBUF:])
        sx, se, so = (sems[:_NBUF], sems[_NBUF:2 * _NBUF],
                      sems[2 * _NBUF:])
        wid = lax.axis_index("s") * _NC + lax.axis_index("c")
        base = wid * rows_per_w
        pltpu.sync_copy(pm_hbm, pm_v)
        pltpu.sync_copy(ps_hbm, ps_v)

        def cin(ci, b):
            r0 = base + ci * _ROW_CHUNK
            return (
                pltpu.make_async_copy(
                    x_hbm.at[pl.ds(r0, _ROW_CHUNK), :], xb[b], sx[b]),
                pltpu.make_async_copy(
                    e_hbm.at[pl.ds(r0, _ROW_CHUNK), :], eb[b], se[b]),
            )

        def cout(ci, b):
            r0 = base + ci * _ROW_CHUNK
            return pltpu.make_async_copy(
                ob[b], o_hbm.at[pl.ds(r0, _ROW_CHUNK), :], so[b])

        def compute(xr, er, outr):
            @plsc.parallel_loop(0, chunk, step=_L, unroll=_UNROLL)
            def _body(i):
                r = i // ncols
                c = i - r * ncols
                xv = xr[r, pl.ds(c, _L)]
                t = xv * hi
                idx = t.astype(jnp.int32)
                gm = plsc.load_gather(pm_v, [idx])
                gs = plsc.load_gather(ps_v, [idx])
                c0m = plsc.bitcast(gm, jnp.float32)
                c1m = plsc.bitcast(gm << 16, jnp.float32)
                c0s = plsc.bitcast(gs, jnp.float32)
                c1s = plsc.bitcast(gs << 16, jnp.float32)
                ev = er[r, pl.ds(c, _L)]
                outr[r, pl.ds(c, _L)] = (
                    (c0m + t * c1m) + (c0s + t * c1s) * ev)

        for p in range(_NBUF - 1):
            for d in cin(p, p):
                d.start()

        def outer(g, _):
            for b in range(_NBUF):
                ci = _NBUF * g + b

                @pl.when(ci + _NBUF - 1 < n_chunks)
                def _():
                    for d in cin(ci + _NBUF - 1, (b + _NBUF - 1) % _NBUF):
                        d.start()

                for d in cin(ci, b):
                    d.wait()

                @pl.when(ci >= _NBUF)
                def _():
                    cout(ci - _NBUF, b).wait()

                compute(xb[b], eb[b], ob[b])
                cout(ci, b).start()
            return 0

        lax.fori_loop(0, n_chunks // _NBUF, outer, 0)
        for p in range(_NBUF):
            cout(n_chunks - _NBUF + p, p).wait()

    return sc_kernel


def _pack_bf16_pair(c0, c1, pad_to=32):
    lb = lax.bitcast_convert_type(
        c1.astype(jnp.bfloat16), jnp.uint16).astype(jnp.uint32)
    b = lax.bitcast_convert_type(c0.astype(jnp.float32), jnp.uint32)
    cand0 = (b & jnp.uint32(0xFFFF0000)) | lb
    cands = jnp.stack([cand0 - jnp.uint32(0x10000), cand0,
                       cand0 + jnp.uint32(0x10000)])
    vals = lax.bitcast_convert_type(cands, jnp.float32)
    best = jnp.argmin(jnp.abs(vals - c0[None, :]), axis=0)
    packed = lax.bitcast_convert_type(
        jnp.take_along_axis(cands, best[None, :], axis=0)[0], jnp.int32)
    pad = jnp.zeros((pad_to - packed.shape[0],), jnp.int32)
    return jnp.concatenate([packed, pad])


def kernel(x, means, stds, epsilon):
    nrows, ncols = x.shape
    num_levels = means.shape[0]
    scale = 1.0 / 4095.0
    m = means.astype(jnp.float32) * scale
    s = stds.astype(jnp.float32) * scale
    zero = jnp.zeros((1,), jnp.float32)
    dm = jnp.concatenate([m[1:] - m[:-1], zero])
    ds = jnp.concatenate([s[1:] - s[:-1], zero])
    f = jnp.arange(num_levels, dtype=jnp.float32)
    pm = _pack_bf16_pair(m - f * dm, dm)
    ps = _pack_bf16_pair(s - f * ds, ds)
    return _make_sc_call(nrows, ncols, float(num_levels - 1))(
        x, epsilon, pm, ps)


# speedup vs baseline: 2.6769x; 2.6769x over previous
import functools

import jax
import jax.numpy as jnp
from jax import lax
from jax.experimental import pallas as pl
from jax.experimental.pallas import tpu as pltpu
from jax.experimental.pallas import tpu_sc as plsc

_NC = 2
_NS = 16
_L = 16
_NW = _NC * _NS

_ROW_CHUNK = 8
_NBUF = 4
_UNROLL = 16


def _make_sc_call(nrows: int, ncols: int, hi: float):
    rows_per_w = nrows // _NW
    n_chunks = rows_per_w // _ROW_CHUNK
    chunk = _ROW_CHUNK * ncols
    mesh = plsc.VectorSubcoreMesh(
        core_axis_name="c", subcore_axis_name="s",
        num_cores=_NC, num_subcores=_NS)

    @functools.partial(
        pl.kernel,
        out_type=jax.ShapeDtypeStruct((nrows, ncols), jnp.float32),
        mesh=mesh,
        compiler_params=pltpu.CompilerParams(
            needs_layout_passes=False, use_tc_tiling_on_sc=True),
        scratch_types=(
            [pltpu.VMEM((32,), jnp.int32)] * 2
            + [pltpu.VMEM((_ROW_CHUNK, ncols), jnp.float32)] * (3 * _NBUF)
            + [pltpu.SemaphoreType.DMA] * (3 * _NBUF)
        ),
    )
    def sc_kernel(x_hbm, e_hbm, pm_hbm, ps_hbm, o_hbm, pm_v, ps_v, *rest):
        bufs = rest[:3 * _NBUF]
        sems = rest[3 * _NBUF:]
        xb, eb, ob = (bufs[:_NBUF], bufs[_NBUF:2 * _NBUF],
                      bufs[2 * _NBUF:])
        sx, se, so = (sems[:_NBUF], sems[_NBUF:2 * _NBUF],
                      sems[2 * _NBUF:])
        wid = lax.axis_index("s") * _NC + lax.axis_index("c")
        base = wid * rows_per_w
        pltpu.sync_copy(pm_hbm, pm_v)
        pltpu.sync_copy(ps_hbm, ps_v)

        def cin(ci, b):
            r0 = base + ci * _ROW_CHUNK
            return (
                pltpu.make_async_copy(
                    x_hbm.at[pl.ds(r0, _ROW_CHUNK), :], xb[b], sx[b]),
                pltpu.make_async_copy(
                    e_hbm.at[pl.ds(r0, _ROW_CHUNK), :], eb[b], se[b]),
            )

        def cout(ci, b):
            r0 = base + ci * _ROW_CHUNK
            return pltpu.make_async_copy(
                ob[b], o_hbm.at[pl.ds(r0, _ROW_CHUNK), :], so[b])

        def compute(xr, er, outr):
            @plsc.parallel_loop(0, chunk, step=_L, unroll=_UNROLL)
            def _body(i):
                r = i // ncols
                c = i - r * ncols
                xv = xr[r, pl.ds(c, _L)]
                t = xv * hi
                idx = t.astype(jnp.int32)
                gm = plsc.load_gather(pm_v, [idx])
                gs = plsc.load_gather(ps_v, [idx])
                c0m = plsc.bitcast(gm, jnp.float32)
                c1m = plsc.bitcast(gm << 16, jnp.float32)
                c0s = plsc.bitcast(gs, jnp.float32)
                c1s = plsc.bitcast(gs << 16, jnp.float32)
                ev = er[r, pl.ds(c, _L)]
                outr[r, pl.ds(c, _L)] = (
                    (c0m + t * c1m) + (c0s + t * c1s) * ev)

        for p in range(_NBUF - 1):
            for d in cin(p, p):
                d.start()

        def outer(g, _):
            for b in range(_NBUF):
                ci = _NBUF * g + b

                @pl.when(ci + _NBUF - 1 < n_chunks)
                def _():
                    for d in cin(ci + _NBUF - 1, (b + _NBUF - 1) % _NBUF):
                        d.start()

                for d in cin(ci, b):
                    d.wait()

                @pl.when(ci >= _NBUF)
                def _():
                    cout(ci - _NBUF, b).wait()

                compute(xb[b], eb[b], ob[b])
                cout(ci, b).start()
            return 0

        lax.fori_loop(0, n_chunks // _NBUF, outer, 0)
        for p in range(_NBUF):
            cout(n_chunks - _NBUF + p, p).wait()

    return sc_kernel


def _pack_bf16_pair(c0, c1, pad_to=32):
    lb = lax.bitcast_convert_type(
        c1.astype(jnp.bfloat16), jnp.uint16).astype(jnp.uint32)
    b = lax.bitcast_convert_type(c0.astype(jnp.float32), jnp.uint32)
    cand0 = (b & jnp.uint32(0xFFFF0000)) | lb
    cands = jnp.stack([cand0 - jnp.uint32(0x10000), cand0,
                       cand0 + jnp.uint32(0x10000)])
    vals = lax.bitcast_convert_type(cands, jnp.float32)
    best = jnp.argmin(jnp.abs(vals - c0[None, :]), axis=0)
    packed = lax.bitcast_convert_type(
        jnp.take_along_axis(cands, best[None, :], axis=0)[0], jnp.int32)
    pad = jnp.zeros((pad_to - packed.shape[0],), jnp.int32)
    return jnp.concatenate([packed, pad])


def kernel(x, means, stds, epsilon):
    nrows, ncols = x.shape
    num_levels = means.shape[0]
    scale = 1.0 / 4095.0
    m = means.astype(jnp.float32) * scale
    s = stds.astype(jnp.float32) * scale
    zero = jnp.zeros((1,), jnp.float32)
    dm = jnp.concatenate([m[1:] - m[:-1], zero])
    ds = jnp.concatenate([s[1:] - s[:-1], zero])
    f = jnp.arange(num_levels, dtype=jnp.float32)
    pm = _pack_bf16_pair(m - f * dm, dm)
    ps = _pack_bf16_pair(s - f * ds, ds)
    return _make_sc_call(nrows, ncols, float(num_levels - 1))(
        x, epsilon, pm, ps)


# speedup vs baseline: 2.6846x; 1.0029x over previous
import functools

import jax
import jax.numpy as jnp
from jax import lax
from jax.experimental import pallas as pl
from jax.experimental.pallas import tpu as pltpu
from jax.experimental.pallas import tpu_sc as plsc

_NC = 2
_NS = 16
_L = 16
_NW = _NC * _NS

_ROW_CHUNK = 16
_NBUF = 2
_UNROLL = 16


def _make_sc_call(nrows: int, ncols: int, hi: float):
    rows_per_w = nrows // _NW
    n_chunks = rows_per_w // _ROW_CHUNK
    chunk = _ROW_CHUNK * ncols
    mesh = plsc.VectorSubcoreMesh(
        core_axis_name="c", subcore_axis_name="s",
        num_cores=_NC, num_subcores=_NS)

    @functools.partial(
        pl.kernel,
        out_type=jax.ShapeDtypeStruct((nrows, ncols), jnp.float32),
        mesh=mesh,
        compiler_params=pltpu.CompilerParams(
            needs_layout_passes=False, use_tc_tiling_on_sc=True),
        scratch_types=(
            [pltpu.VMEM((32,), jnp.int32)] * 2
            + [pltpu.VMEM((_ROW_CHUNK, ncols), jnp.float32)] * (3 * _NBUF)
            + [pltpu.SemaphoreType.DMA] * (3 * _NBUF)
        ),
    )
    def sc_kernel(x_hbm, e_hbm, pm_hbm, ps_hbm, o_hbm, pm_v, ps_v, *rest):
        bufs = rest[:3 * _NBUF]
        sems = rest[3 * _NBUF:]
        xb, eb, ob = (bufs[:_NBUF], bufs[_NBUF:2 * _NBUF],
                      bufs[2 * _NBUF:])
        sx, se, so = (sems[:_NBUF], sems[_NBUF:2 * _NBUF],
                      sems[2 * _NBUF:])
        wid = lax.axis_index("s") * _NC + lax.axis_index("c")
        base = wid * rows_per_w
        pltpu.sync_copy(pm_hbm, pm_v)
        pltpu.sync_copy(ps_hbm, ps_v)

        def cin(ci, b):
            r0 = base + ci * _ROW_CHUNK
            return (
                pltpu.make_async_copy(
                    x_hbm.at[pl.ds(r0, _ROW_CHUNK), :], xb[b], sx[b]),
                pltpu.make_async_copy(
                    e_hbm.at[pl.ds(r0, _ROW_CHUNK), :], eb[b], se[b]),
            )

        def cout(ci, b):
            r0 = base + ci * _ROW_CHUNK
            return pltpu.make_async_copy(
                ob[b], o_hbm.at[pl.ds(r0, _ROW_CHUNK), :], so[b])

        def compute(xr, er, outr):
            @plsc.parallel_loop(0, chunk, step=_L, unroll=_UNROLL)
            def _body(i):
                r = i // ncols
                c = i - r * ncols
                xv = xr[r, pl.ds(c, _L)]
                t = xv * hi
                idx = t.astype(jnp.int32)
                gm = plsc.load_gather(pm_v, [idx])
                gs = plsc.load_gather(ps_v, [idx])
                c0m = plsc.bitcast(gm, jnp.float32)
                c1m = plsc.bitcast(gm << 16, jnp.float32)
                c0s = plsc.bitcast(gs, jnp.float32)
                c1s = plsc.bitcast(gs << 16, jnp.float32)
                ev = er[r, pl.ds(c, _L)]
                outr[r, pl.ds(c, _L)] = (
                    (c0m + t * c1m) + (c0s + t * c1s) * ev)

        for p in range(_NBUF - 1):
            for d in cin(p, p):
                d.start()

        def outer(g, _):
            for b in range(_NBUF):
                ci = _NBUF * g + b

                @pl.when(ci + _NBUF - 1 < n_chunks)
                def _():
                    for d in cin(ci + _NBUF - 1, (b + _NBUF - 1) % _NBUF):
                        d.start()

                for d in cin(ci, b):
                    d.wait()

                @pl.when(ci >= _NBUF)
                def _():
                    cout(ci - _NBUF, b).wait()

                compute(xb[b], eb[b], ob[b])
                cout(ci, b).start()
            return 0

        lax.fori_loop(0, n_chunks // _NBUF, outer, 0)
        for p in range(_NBUF):
            cout(n_chunks - _NBUF + p, p).wait()

    return sc_kernel


def _pack_bf16_pair(c0, c1, pad_to=32):
    lb = lax.bitcast_convert_type(
        c1.astype(jnp.bfloat16), jnp.uint16).astype(jnp.uint32)
    b = lax.bitcast_convert_type(c0.astype(jnp.float32), jnp.uint32)
    cand0 = (b & jnp.uint32(0xFFFF0000)) | lb
    cands = jnp.stack([cand0 - jnp.uint32(0x10000), cand0,
                       cand0 + jnp.uint32(0x10000)])
    vals = lax.bitcast_convert_type(cands, jnp.float32)
    best = jnp.argmin(jnp.abs(vals - c0[None, :]), axis=0)
    packed = lax.bitcast_convert_type(
        jnp.take_along_axis(cands, best[None, :], axis=0)[0], jnp.int32)
    pad = jnp.zeros((pad_to - packed.shape[0],), jnp.int32)
    return jnp.concatenate([packed, pad])


def kernel(x, means, stds, epsilon):
    nrows, ncols = x.shape
    num_levels = means.shape[0]
    scale = 1.0 / 4095.0
    m = means.astype(jnp.float32) * scale
    s = stds.astype(jnp.float32) * scale
    zero = jnp.zeros((1,), jnp.float32)
    dm = jnp.concatenate([m[1:] - m[:-1], zero])
    ds = jnp.concatenate([s[1:] - s[:-1], zero])
    f = jnp.arange(num_levels, dtype=jnp.float32)
    pm = _pack_bf16_pair(m - f * dm, dm)
    ps = _pack_bf16_pair(s - f * ds, ds)
    return _make_sc_call(nrows, ncols, float(num_levels - 1))(
        x, epsilon, pm, ps)
